# baseline (speedup 1.0000x reference)
import functools

import jax
import jax.numpy as jnp
import numpy as np
from jax import lax
from jax.experimental import pallas as pl
from jax.experimental.pallas import tpu as pltpu
from jax.experimental.pallas import tpu_sc as plsc

N_NODES = 10000
N_EDGES = 320000
NBASIS = 20
NHID = 100
RCUT = 5.0
NUM_NEIGH = 32.0
NGRAPH = 256
LAYER_DIMS = [(128, 40), (40, 40), (40, 40), (40, 64)]

NC = 2
NS = 16
NW = NC * NS
CHUNK = 80


def _normalize2mom_silu():
    z = np.linspace(-12.0, 12.0, 240001)
    pdf = np.exp(-0.5 * z * z) / np.sqrt(2.0 * np.pi)
    s = z / (1.0 + np.exp(-z))
    m2 = np.trapz(s * s * pdf, z)
    return float(m2 ** -0.5)


SILU_C = _normalize2mom_silu()


def _sc_gather(table, idx):
    n_rows, d = table.shape
    n_idx = idx.shape[0]
    per_w = n_idx // NW
    n_ch = per_w // CHUNK
    mesh = plsc.VectorSubcoreMesh(core_axis_name="c", subcore_axis_name="s")

    @functools.partial(
        pl.kernel,
        out_type=jax.ShapeDtypeStruct((n_idx, d), jnp.float32),
        mesh=mesh,
        scratch_types=[
            pltpu.VMEM((CHUNK,), jnp.int32),
            pltpu.VMEM((CHUNK, d), jnp.float32),
            pltpu.SemaphoreType.DMA,
        ],
    )
    def k(table_hbm, idx_hbm, out_hbm, idx_v, rows_v, sem):
        wid = lax.axis_index("s") * NC + lax.axis_index("c")
        base = wid * per_w

        def body(j, carry):
            off = base + j * CHUNK
            pltpu.sync_copy(idx_hbm.at[pl.ds(off, CHUNK)], idx_v)
            pltpu.async_copy(table_hbm.at[idx_v], rows_v, sem).wait()
            pltpu.sync_copy(rows_v, out_hbm.at[pl.ds(off, CHUNK)])
            return carry

        lax.fori_loop(0, n_ch, body, 0)

    return k(table, idx)


def _sc_scatter_add(rows, idx, n_out):
    n_rows, d = rows.shape
    per_w = n_rows // NW
    n_ch = per_w // CHUNK
    zeros = jnp.zeros((n_out, d), jnp.float32)
    mesh = plsc.VectorSubcoreMesh(core_axis_name="c", subcore_axis_name="s")

    @functools.partial(
        pl.kernel,
        out_type=jax.ShapeDtypeStruct((NC, n_out, d), jnp.float32),
        mesh=mesh,
        scratch_types=[
            pltpu.VMEM((CHUNK,), jnp.int32),
            pltpu.VMEM((CHUNK, d), jnp.float32),
            pltpu.VMEM_SHARED((n_out, d), jnp.float32),
        ],
    )
    def k(rows_hbm, idx_hbm, zeros_hbm, out_hbm, idx_v, rows_v, accum):
        cid = lax.axis_index("c")
        sid = lax.axis_index("s")
        wid = sid * NC + cid

        @pl.when(sid == 0)
        def _():
            pltpu.sync_copy(zeros_hbm, accum)

        plsc.subcore_barrier()

        def body(j, carry):
            off = wid * per_w + j * CHUNK
            pltpu.sync_copy(idx_hbm.at[pl.ds(off, CHUNK)], idx_v)
            pltpu.sync_copy(rows_hbm.at[pl.ds(off, CHUNK)], rows_v)
            pltpu.sync_copy(rows_v, accum.at[idx_v], add=True)
            return carry

        lax.fori_loop(0, n_ch, body, 0)
        plsc.subcore_barrier()

        @pl.when(sid == 0)
        def _():
            pltpu.sync_copy(accum, out_hbm.at[cid])

    return k(rows, idx, zeros)


_BLK = 256


def _emb_kernel(pgs_ref, pgd_ref, out_ref):
    step = np.float32(RCUT / (NBASIS + 1))
    vals = (lax.broadcasted_iota(jnp.int32, (1, NBASIS), 1)
            .astype(jnp.float32) + 1.0) * step
    vec = pgd_ref[:, :3] - pgs_ref[:, :3]
    d2 = jnp.sum(vec * vec, axis=1, keepdims=True)
    dist = jnp.sqrt(d2 + 1e-12)
    diff = (dist - vals) * (1.0 / step)

    def sus(t):
        safe = jnp.where(t > 0.0, t, 1.0)
        return jnp.where(t > 0.0, jnp.exp(-1.0 / safe), 0.0)

    c = np.float32(1.14136 * np.exp(2.0))
    out_ref[...] = c * sus(diff + 1.0) * sus(1.0 - diff)


def _tc_emb(pgs, pgd):
    e = pgs.shape[0]
    grid = e // _BLK
    return pl.pallas_call(
        _emb_kernel,
        grid=(grid,),
        in_specs=[
            pl.BlockSpec((_BLK, 128), lambda i: (i, 0)),
            pl.BlockSpec((_BLK, 128), lambda i: (i, 0)),
        ],
        out_specs=pl.BlockSpec((_BLK, NBASIS), lambda i: (i, 0)),
        out_shape=jax.ShapeDtypeStruct((e, NBASIS), jnp.float32),
    )(pgs, pgd)


def _layer_kernel(nin, nout, pad_to, activate, emb_ref, g0_ref, g1_ref,
                  w1_ref, w2_ref, out_ref):
    emb = emb_ref[...]
    h = jnp.dot(emb, w1_ref[...], preferred_element_type=jnp.float32)
    h = jax.nn.silu(h) * np.float32(SILU_C)
    tpw = jnp.dot(h, w2_ref[...], preferred_element_type=jnp.float32)
    if g1_ref is None:
        xe = g0_ref[...]
    else:
        xe = g0_ref[...] + g1_ref[...]
    if activate:
        xe = jax.nn.silu(xe) * np.float32(SILU_C)
    xe = xe[:, :nin]
    acc = jnp.zeros((emb.shape[0], nout), jnp.float32)
    for i in range(nin):
        acc = acc + xe[:, i:i + 1] * tpw[:, i * nout:(i + 1) * nout]
    if pad_to > nout:
        acc = jnp.concatenate(
            [acc, jnp.zeros((emb.shape[0], pad_to - nout), jnp.float32)],
            axis=1)
    out_ref[...] = acc


def _tc_layer(emb, gs, w1s, w2s, nin, nout, pad_to, activate):
    e = emb.shape[0]
    grid = e // _BLK
    din = gs[0].shape[1]
    body = functools.partial(_layer_kernel, nin, nout, pad_to, activate)
    if len(gs) == 1:
        body2 = lambda a, b, c, d, o: body(a, b, None, c, d, o)
        in_specs = [
            pl.BlockSpec((_BLK, NBASIS), lambda i: (i, 0)),
            pl.BlockSpec((_BLK, din), lambda i: (i, 0)),
            pl.BlockSpec(w1s.shape, lambda i: (0, 0)),
            pl.BlockSpec(w2s.shape, lambda i: (0, 0)),
        ]
        args = (emb, gs[0], w1s, w2s)
    else:
        body2 = body
        in_specs = [
            pl.BlockSpec((_BLK, NBASIS), lambda i: (i, 0)),
            pl.BlockSpec((_BLK, din), lambda i: (i, 0)),
            pl.BlockSpec((_BLK, din), lambda i: (i, 0)),
            pl.BlockSpec(w1s.shape, lambda i: (0, 0)),
            pl.BlockSpec(w2s.shape, lambda i: (0, 0)),
        ]
        args = (emb, gs[0], gs[1], w1s, w2s)
    return pl.pallas_call(
        body2,
        grid=(grid,),
        in_specs=in_specs,
        out_specs=pl.BlockSpec((_BLK, pad_to), lambda i: (i, 0)),
        out_shape=jax.ShapeDtypeStruct((e, pad_to), jnp.float32),
    )(*args)


def _combine_kernel(q_ref, out_ref):
    out_ref[...] = (q_ref[0] + q_ref[1])[:, :64]


def _tc_combine(q):
    return pl.pallas_call(
        _combine_kernel,
        out_shape=jax.ShapeDtypeStruct((q.shape[1], 64), jnp.float32),
    )(q)


def kernel(x, pos, edge_index, batch, W1_0, W2_0, W1_1, W2_1, W1_2, W2_2,
           W1_3, W2_3):
    f32 = jnp.float32
    x = x.astype(f32)
    src = edge_index[0].astype(jnp.int32)
    dst = edge_index[1].astype(jnp.int32)

    ws = []
    for (w1, w2), (nin, nout) in zip(
            [(W1_0, W2_0), (W1_1, W2_1), (W1_2, W2_2), (W1_3, W2_3)],
            LAYER_DIMS):
        w1s = (w1 / np.sqrt(NBASIS)).astype(f32)
        w2s = (w2 / (np.sqrt(NHID) * np.sqrt(nin) * np.sqrt(NUM_NEIGH))
               ).astype(f32)
        ws.append((w1s, w2s))

    pos_pad = jnp.pad(pos.astype(f32), ((0, 0), (0, 125)))

    pgs = _sc_gather(pos_pad, src)
    pgd = _sc_gather(pos_pad, dst)
    emb = _tc_emb(pgs, pgd)

    gx = _sc_gather(x, dst)
    ef = _tc_layer(emb, [gx], ws[0][0], ws[0][1], 128, 40, 128, False)
    part = _sc_scatter_add(ef, src, N_NODES)

    for l in (1, 2, 3):
        nin, nout = LAYER_DIMS[l]
        g0 = _sc_gather(part[0], dst)
        g1 = _sc_gather(part[1], dst)
        ef = _tc_layer(emb, [g0, g1], ws[l][0], ws[l][1], nin, nout,
                       128, True)
        part = _sc_scatter_add(ef, src, N_NODES)

    rows = part.reshape(2 * N_NODES, 128)
    pad_rows = 2 * N_NODES % (NW * CHUNK)
    pad_rows = NW * CHUNK - pad_rows if pad_rows else 0
    rows = jnp.pad(rows, ((0, pad_rows), (0, 0)))
    bidx = jnp.concatenate([batch.astype(jnp.int32)] * 2 +
                           [jnp.zeros((pad_rows,), jnp.int32)])
    q = _sc_scatter_add(rows, bidx, NGRAPH)
    return _tc_combine(q)


# speedup vs baseline: 1.0161x; 1.0161x over previous
import functools

import jax
import jax.numpy as jnp
import numpy as np
from jax import lax
from jax.experimental import pallas as pl
from jax.experimental.pallas import tpu as pltpu
from jax.experimental.pallas import tpu_sc as plsc

N_NODES = 10000
N_EDGES = 320000
NBASIS = 20
NHID = 100
RCUT = 5.0
NUM_NEIGH = 32.0
NGRAPH = 256
LAYER_DIMS = [(128, 40), (40, 40), (40, 40), (40, 64)]

NC = 2
NS = 16
NW = NC * NS
CHUNK = 80


def _normalize2mom_silu():
    z = np.linspace(-12.0, 12.0, 240001)
    pdf = np.exp(-0.5 * z * z) / np.sqrt(2.0 * np.pi)
    s = z / (1.0 + np.exp(-z))
    m2 = np.trapz(s * s * pdf, z)
    return float(m2 ** -0.5)


SILU_C = _normalize2mom_silu()


def _sc_gather(table, idx):
    n_rows, d = table.shape
    n_idx = idx.shape[0]
    per_w = n_idx // NW
    n_ch = per_w // CHUNK
    mesh = plsc.VectorSubcoreMesh(core_axis_name="c", subcore_axis_name="s")

    @functools.partial(
        pl.kernel,
        out_type=jax.ShapeDtypeStruct((n_idx, d), jnp.float32),
        mesh=mesh,
        scratch_types=[
            pltpu.VMEM((CHUNK,), jnp.int32),
            pltpu.VMEM((CHUNK, d), jnp.float32),
            pltpu.SemaphoreType.DMA,
        ],
    )
    def k(table_hbm, idx_hbm, out_hbm, idx_v, rows_v, sem):
        wid = lax.axis_index("s") * NC + lax.axis_index("c")
        base = wid * per_w

        def body(j, carry):
            off = base + j * CHUNK
            pltpu.sync_copy(idx_hbm.at[pl.ds(off, CHUNK)], idx_v)
            pltpu.async_copy(table_hbm.at[idx_v], rows_v, sem).wait()
            pltpu.sync_copy(rows_v, out_hbm.at[pl.ds(off, CHUNK)])
            return carry

        lax.fori_loop(0, n_ch, body, 0)

    return k(table, idx)


def _sc_scatter_add(rows, idx, n_out):
    n_rows, d = rows.shape
    per_w = n_rows // NW
    n_ch = per_w // CHUNK
    zeros = jnp.zeros((n_out, d), jnp.float32)
    mesh = plsc.VectorSubcoreMesh(core_axis_name="c", subcore_axis_name="s")

    @functools.partial(
        pl.kernel,
        out_type=jax.ShapeDtypeStruct((NC, n_out, d), jnp.float32),
        mesh=mesh,
        scratch_types=[
            pltpu.VMEM((CHUNK,), jnp.int32),
            pltpu.VMEM((CHUNK, d), jnp.float32),
            pltpu.VMEM_SHARED((n_out, d), jnp.float32),
        ],
    )
    def k(rows_hbm, idx_hbm, zeros_hbm, out_hbm, idx_v, rows_v, accum):
        cid = lax.axis_index("c")
        sid = lax.axis_index("s")
        wid = sid * NC + cid

        @pl.when(sid == 0)
        def _():
            pltpu.sync_copy(zeros_hbm, accum)

        plsc.subcore_barrier()

        def body(j, carry):
            off = wid * per_w + j * CHUNK
            pltpu.sync_copy(idx_hbm.at[pl.ds(off, CHUNK)], idx_v)
            pltpu.sync_copy(rows_hbm.at[pl.ds(off, CHUNK)], rows_v)
            pltpu.sync_copy(rows_v, accum.at[idx_v], add=True)
            return carry

        lax.fori_loop(0, n_ch, body, 0)
        plsc.subcore_barrier()

        @pl.when(sid == 0)
        def _():
            pltpu.sync_copy(accum, out_hbm.at[cid])

    return k(rows, idx, zeros)


_BLK = 256


def _emb_kernel(pgs_ref, pgd_ref, out_ref):
    step = np.float32(RCUT / (NBASIS + 1))
    vals = (lax.broadcasted_iota(jnp.int32, (1, NBASIS), 1)
            .astype(jnp.float32) + 1.0) * step
    vec = pgd_ref[:, :3] - pgs_ref[:, :3]
    d2 = jnp.sum(vec * vec, axis=1, keepdims=True)
    dist = jnp.sqrt(d2 + 1e-12)
    diff = (dist - vals) * (1.0 / step)

    def sus(t):
        safe = jnp.where(t > 0.0, t, 1.0)
        return jnp.where(t > 0.0, jnp.exp(-1.0 / safe), 0.0)

    c = np.float32(1.14136 * np.exp(2.0))
    out_ref[...] = c * sus(diff + 1.0) * sus(1.0 - diff)


def _tc_emb(pgs, pgd):
    e = pgs.shape[0]
    grid = e // _BLK
    return pl.pallas_call(
        _emb_kernel,
        grid=(grid,),
        in_specs=[
            pl.BlockSpec((_BLK, 128), lambda i: (i, 0)),
            pl.BlockSpec((_BLK, 128), lambda i: (i, 0)),
        ],
        out_specs=pl.BlockSpec((_BLK, NBASIS), lambda i: (i, 0)),
        out_shape=jax.ShapeDtypeStruct((e, NBASIS), jnp.float32),
    )(pgs, pgd)


def _layer_kernel(nin, nout, pad_to, packed, emb_ref, g_ref,
                  w1_ref, w2_ref, out_ref):
    emb = emb_ref[...]
    h = jnp.dot(emb, w1_ref[...], preferred_element_type=jnp.float32)
    h = jax.nn.silu(h) * np.float32(SILU_C)
    tpw = jnp.dot(h, w2_ref[...], preferred_element_type=jnp.float32)
    g = g_ref[...]
    if packed:
        xe = g[:, :nin] + g[:, 64:64 + nin]
        xe = jax.nn.silu(xe) * np.float32(SILU_C)
    else:
        xe = g[:, :nin]
    acc = jnp.zeros((emb.shape[0], nout), jnp.float32)
    for i in range(nin):
        acc = acc + xe[:, i:i + 1] * tpw[:, i * nout:(i + 1) * nout]
    if pad_to > nout:
        acc = jnp.concatenate(
            [acc, jnp.zeros((emb.shape[0], pad_to - nout), jnp.float32)],
            axis=1)
    out_ref[...] = acc


def _tc_layer(emb, g, w1s, w2s, nin, nout, pad_to, packed):
    e = emb.shape[0]
    grid = e // _BLK
    din = g.shape[1]
    body = functools.partial(_layer_kernel, nin, nout, pad_to, packed)
    in_specs = [
        pl.BlockSpec((_BLK, NBASIS), lambda i: (i, 0)),
        pl.BlockSpec((_BLK, din), lambda i: (i, 0)),
        pl.BlockSpec(w1s.shape, lambda i: (0, 0)),
        pl.BlockSpec(w2s.shape, lambda i: (0, 0)),
    ]
    return pl.pallas_call(
        body,
        grid=(grid,),
        in_specs=in_specs,
        out_specs=pl.BlockSpec((_BLK, pad_to), lambda i: (i, 0)),
        out_shape=jax.ShapeDtypeStruct((e, pad_to), jnp.float32),
    )(emb, g, w1s, w2s)


def _pack_kernel(p_ref, out_ref):
    out_ref[...] = jnp.concatenate(
        [p_ref[0, :, :64], p_ref[1, :, :64]], axis=1)


def _tc_pack(part):
    n = part.shape[1]
    return pl.pallas_call(
        _pack_kernel,
        out_shape=jax.ShapeDtypeStruct((n, 128), jnp.float32),
    )(part)


def _combine_kernel(q_ref, out_ref):
    out_ref[...] = (q_ref[0] + q_ref[1])[:, :64]


def _tc_combine(q):
    return pl.pallas_call(
        _combine_kernel,
        out_shape=jax.ShapeDtypeStruct((q.shape[1], 64), jnp.float32),
    )(q)


def kernel(x, pos, edge_index, batch, W1_0, W2_0, W1_1, W2_1, W1_2, W2_2,
           W1_3, W2_3):
    f32 = jnp.float32
    x = x.astype(f32)
    src = edge_index[0].astype(jnp.int32)
    dst = edge_index[1].astype(jnp.int32)

    ws = []
    for (w1, w2), (nin, nout) in zip(
            [(W1_0, W2_0), (W1_1, W2_1), (W1_2, W2_2), (W1_3, W2_3)],
            LAYER_DIMS):
        w1s = (w1 / np.sqrt(NBASIS)).astype(f32)
        w2s = (w2 / (np.sqrt(NHID) * np.sqrt(nin) * np.sqrt(NUM_NEIGH))
               ).astype(f32)
        ws.append((w1s, w2s))

    pos_pad = jnp.pad(pos.astype(f32), ((0, 0), (0, 125)))

    pgs = _sc_gather(pos_pad, src)
    pgd = _sc_gather(pos_pad, dst)
    emb = _tc_emb(pgs, pgd)

    gx = _sc_gather(x, dst)
    ef = _tc_layer(emb, gx, ws[0][0], ws[0][1], 128, 40, 128, False)
    part = _sc_scatter_add(ef, src, N_NODES)

    for l in (1, 2, 3):
        nin, nout = LAYER_DIMS[l]
        packed = _tc_pack(part)
        g = _sc_gather(packed, dst)
        ef = _tc_layer(emb, g, ws[l][0], ws[l][1], nin, nout, 128, True)
        part = _sc_scatter_add(ef, src, N_NODES)

    rows = part.reshape(2 * N_NODES, 128)
    pad_rows = 2 * N_NODES % (NW * CHUNK)
    pad_rows = NW * CHUNK - pad_rows if pad_rows else 0
    rows = jnp.pad(rows, ((0, pad_rows), (0, 0)))
    bidx = jnp.concatenate([batch.astype(jnp.int32)] * 2 +
                           [jnp.zeros((pad_rows,), jnp.int32)])
    q = _sc_scatter_add(rows, bidx, NGRAPH)
    return _tc_combine(q)


# speedup vs baseline: 2.2924x; 2.2561x over previous
import functools

import jax
import jax.numpy as jnp
import numpy as np
from jax import lax
from jax.experimental import pallas as pl
from jax.experimental.pallas import tpu as pltpu
from jax.experimental.pallas import tpu_sc as plsc

N_NODES = 10000
N_EDGES = 320000
NBASIS = 20
NHID = 100
RCUT = 5.0
NUM_NEIGH = 32.0
NGRAPH = 256
LAYER_DIMS = [(128, 40), (40, 40), (40, 40), (40, 64)]

NC = 2
NS = 16
NW = NC * NS
CHUNK = 80


def _normalize2mom_silu():
    z = np.linspace(-12.0, 12.0, 240001)
    pdf = np.exp(-0.5 * z * z) / np.sqrt(2.0 * np.pi)
    s = z / (1.0 + np.exp(-z))
    m2 = np.trapz(s * s * pdf, z)
    return float(m2 ** -0.5)


SILU_C = _normalize2mom_silu()


def _sc_gather(table, idx):
    n_rows, d = table.shape
    n_idx = idx.shape[0]
    per_w = n_idx // NW
    n_ch = per_w // CHUNK
    mesh = plsc.VectorSubcoreMesh(core_axis_name="c", subcore_axis_name="s")

    @functools.partial(
        pl.kernel,
        out_type=jax.ShapeDtypeStruct((n_idx, d), jnp.float32),
        mesh=mesh,
        scratch_types=[
            pltpu.VMEM((CHUNK,), jnp.int32),
            pltpu.VMEM((CHUNK, d), jnp.float32),
            pltpu.SemaphoreType.DMA,
        ],
    )
    def k(table_hbm, idx_hbm, out_hbm, idx_v, rows_v, sem):
        wid = lax.axis_index("s") * NC + lax.axis_index("c")
        base = wid * per_w

        def body(j, carry):
            off = base + j * CHUNK
            pltpu.sync_copy(idx_hbm.at[pl.ds(off, CHUNK)], idx_v)
            pltpu.async_copy(table_hbm.at[idx_v], rows_v, sem).wait()
            pltpu.sync_copy(rows_v, out_hbm.at[pl.ds(off, CHUNK)])
            return carry

        lax.fori_loop(0, n_ch, body, 0)

    return k(table, idx)


def _sc_scatter_add(rows, idx, n_out):
    n_rows, d = rows.shape
    per_w = n_rows // NW
    n_ch = per_w // CHUNK
    zeros = jnp.zeros((n_out, d), jnp.float32)
    mesh = plsc.VectorSubcoreMesh(core_axis_name="c", subcore_axis_name="s")

    @functools.partial(
        pl.kernel,
        out_type=jax.ShapeDtypeStruct((NC, n_out, d), jnp.float32),
        mesh=mesh,
        scratch_types=[
            pltpu.VMEM((CHUNK,), jnp.int32),
            pltpu.VMEM((CHUNK, d), jnp.float32),
            pltpu.VMEM_SHARED((n_out, d), jnp.float32),
        ],
    )
    def k(rows_hbm, idx_hbm, zeros_hbm, out_hbm, idx_v, rows_v, accum):
        cid = lax.axis_index("c")
        sid = lax.axis_index("s")
        wid = sid * NC + cid

        @pl.when(sid == 0)
        def _():
            pltpu.sync_copy(zeros_hbm, accum)

        plsc.subcore_barrier()

        def body(j, carry):
            off = wid * per_w + j * CHUNK
            pltpu.sync_copy(idx_hbm.at[pl.ds(off, CHUNK)], idx_v)
            pltpu.sync_copy(rows_hbm.at[pl.ds(off, CHUNK)], rows_v)
            pltpu.sync_copy(rows_v, accum.at[idx_v], add=True)
            return carry

        lax.fori_loop(0, n_ch, body, 0)
        plsc.subcore_barrier()

        @pl.when(sid == 0)
        def _():
            pltpu.sync_copy(accum, out_hbm.at[cid])

    return k(rows, idx, zeros)


_BLK = 256


def _emb_kernel(pgs_ref, pgd_ref, out_ref):
    step = np.float32(RCUT / (NBASIS + 1))
    vals = (lax.broadcasted_iota(jnp.int32, (1, NBASIS), 1)
            .astype(jnp.float32) + 1.0) * step
    vec = pgd_ref[:, :3] - pgs_ref[:, :3]
    d2 = jnp.sum(vec * vec, axis=1, keepdims=True)
    dist = jnp.sqrt(d2 + 1e-12)
    diff = (dist - vals) * (1.0 / step)

    def sus(t):
        safe = jnp.where(t > 0.0, t, 1.0)
        return jnp.where(t > 0.0, jnp.exp(-1.0 / safe), 0.0)

    c = np.float32(1.14136 * np.exp(2.0))
    out_ref[...] = c * sus(diff + 1.0) * sus(1.0 - diff)


def _tc_emb(pgs, pgd):
    e = pgs.shape[0]
    grid = e // _BLK
    return pl.pallas_call(
        _emb_kernel,
        grid=(grid,),
        in_specs=[
            pl.BlockSpec((_BLK, 128), lambda i: (i, 0)),
            pl.BlockSpec((_BLK, 128), lambda i: (i, 0)),
        ],
        out_specs=pl.BlockSpec((_BLK, NBASIS), lambda i: (i, 0)),
        out_shape=jax.ShapeDtypeStruct((e, NBASIS), jnp.float32),
    )(pgs, pgd)


def _layer_kernel(nin, nout, pad_to, packed, emb_ref, g_ref,
                  w1_ref, w2_ref, out_ref):
    emb = emb_ref[...]
    h = jnp.dot(emb, w1_ref[...], preferred_element_type=jnp.float32)
    h = jax.nn.silu(h) * np.float32(SILU_C)
    tpw = jnp.dot(h, w2_ref[...], preferred_element_type=jnp.float32)
    g = g_ref[...]
    if packed:
        xe = g[:, :nin] + g[:, 64:64 + nin]
        xe = jax.nn.silu(xe) * np.float32(SILU_C)
    else:
        xe = g[:, :nin]
    acc = jnp.zeros((emb.shape[0], 128), jnp.float32)
    for i in range(nin):
        acc = acc + xe[:, i:i + 1] * tpw[:, i * 128:(i + 1) * 128]
    out_ref[...] = acc


def _tc_layer(emb, g, w1s, w2s, nin, nout, blk, packed):
    e = emb.shape[0]
    grid = e // blk
    din = g.shape[1]
    body = functools.partial(_layer_kernel, nin, nout, 128, packed)
    in_specs = [
        pl.BlockSpec((blk, NBASIS), lambda i: (i, 0)),
        pl.BlockSpec((blk, din), lambda i: (i, 0)),
        pl.BlockSpec(w1s.shape, lambda i: (0, 0)),
        pl.BlockSpec(w2s.shape, lambda i: (0, 0)),
    ]
    return pl.pallas_call(
        body,
        grid=(grid,),
        in_specs=in_specs,
        out_specs=pl.BlockSpec((blk, 128), lambda i: (i, 0)),
        out_shape=jax.ShapeDtypeStruct((e, 128), jnp.float32),
    )(emb, g, w1s, w2s)


def _pack_kernel(p_ref, out_ref):
    out_ref[...] = jnp.concatenate(
        [p_ref[0, :, :64], p_ref[1, :, :64]], axis=1)


def _tc_pack(part):
    n = part.shape[1]
    return pl.pallas_call(
        _pack_kernel,
        out_shape=jax.ShapeDtypeStruct((n, 128), jnp.float32),
    )(part)


def _combine_kernel(q_ref, out_ref):
    out_ref[...] = (q_ref[0] + q_ref[1])[:, :64]


def _tc_combine(q):
    return pl.pallas_call(
        _combine_kernel,
        out_shape=jax.ShapeDtypeStruct((q.shape[1], 64), jnp.float32),
    )(q)


def kernel(x, pos, edge_index, batch, W1_0, W2_0, W1_1, W2_1, W1_2, W2_2,
           W1_3, W2_3):
    f32 = jnp.float32
    x = x.astype(f32)
    src = edge_index[0].astype(jnp.int32)
    dst = edge_index[1].astype(jnp.int32)

    ws = []
    for (w1, w2), (nin, nout) in zip(
            [(W1_0, W2_0), (W1_1, W2_1), (W1_2, W2_2), (W1_3, W2_3)],
            LAYER_DIMS):
        w1s = (w1 / np.sqrt(NBASIS)).astype(f32)
        w2s = (w2 / (np.sqrt(NHID) * np.sqrt(nin) * np.sqrt(NUM_NEIGH))
               ).astype(f32)
        w2p = jnp.pad(w2s.reshape(NHID, nin, nout),
                      ((0, 0), (0, 0), (0, 128 - nout)))
        ws.append((w1s, w2p.reshape(NHID, nin * 128)))

    pos_pad = jnp.pad(pos.astype(f32), ((0, 0), (0, 125)))

    pgs = _sc_gather(pos_pad, src)
    pgd = _sc_gather(pos_pad, dst)
    emb = _tc_emb(pgs, pgd)

    gx = _sc_gather(x, dst)
    ef = _tc_layer(emb, gx, ws[0][0], ws[0][1], 128, 40, 128, False)
    part = _sc_scatter_add(ef, src, N_NODES)

    for l in (1, 2, 3):
        nin, nout = LAYER_DIMS[l]
        packed = _tc_pack(part)
        g = _sc_gather(packed, dst)
        ef = _tc_layer(emb, g, ws[l][0], ws[l][1], nin, nout, 256, True)
        part = _sc_scatter_add(ef, src, N_NODES)

    rows = part.reshape(2 * N_NODES, 128)
    pad_rows = 2 * N_NODES % (NW * CHUNK)
    pad_rows = NW * CHUNK - pad_rows if pad_rows else 0
    rows = jnp.pad(rows, ((0, pad_rows), (0, 0)))
    bidx = jnp.concatenate([batch.astype(jnp.int32)] * 2 +
                           [jnp.zeros((pad_rows,), jnp.int32)])
    q = _sc_scatter_add(rows, bidx, NGRAPH)
    return _tc_combine(q)


# speedup vs baseline: 2.4016x; 1.0476x over previous
import functools

import jax
import jax.numpy as jnp
import numpy as np
from jax import lax
from jax.experimental import pallas as pl
from jax.experimental.pallas import tpu as pltpu
from jax.experimental.pallas import tpu_sc as plsc

N_NODES = 10000
N_EDGES = 320000
NBASIS = 20
NHID = 100
RCUT = 5.0
NUM_NEIGH = 32.0
NGRAPH = 256
LAYER_DIMS = [(128, 40), (40, 40), (40, 40), (40, 64)]

NC = 2
NS = 16
NW = NC * NS
CHUNK = 80


def _normalize2mom_silu():
    z = np.linspace(-12.0, 12.0, 240001)
    pdf = np.exp(-0.5 * z * z) / np.sqrt(2.0 * np.pi)
    s = z / (1.0 + np.exp(-z))
    m2 = np.trapz(s * s * pdf, z)
    return float(m2 ** -0.5)


SILU_C = _normalize2mom_silu()


def _pipe_k(n_ch):
    for cand in (5, 4, 2):
        if n_ch % cand == 0:
            return cand
    return 1


def _sc_gather(table, idx):
    n_rows, d = table.shape
    n_idx = idx.shape[0]
    per_w = n_idx // NW
    n_ch = per_w // CHUNK
    kk = _pipe_k(n_ch)
    idx2d = idx.reshape(NW, n_ch, CHUNK)
    mesh = plsc.VectorSubcoreMesh(core_axis_name="c", subcore_axis_name="s")

    @functools.partial(
        pl.kernel,
        out_type=jax.ShapeDtypeStruct((n_idx, d), jnp.float32),
        mesh=mesh,
        scratch_types=[
            pltpu.VMEM((n_ch, CHUNK), jnp.int32),
            pltpu.VMEM((kk, CHUNK, d), jnp.float32),
            pltpu.SemaphoreType.DMA,
        ],
    )
    def k(table_hbm, idx_hbm, out_hbm, idx_v, rows_v, sem):
        wid = lax.axis_index("s") * NC + lax.axis_index("c")
        base = wid * per_w
        pltpu.sync_copy(idx_hbm.at[wid], idx_v)

        def body(b, carry):
            j0 = b * kk
            hs = []
            for t in range(kk):
                hs.append(pltpu.async_copy(
                    table_hbm.at[idx_v.at[j0 + t]], rows_v.at[t], sem))
            for t in range(kk):
                hs[t].wait()
                pltpu.sync_copy(
                    rows_v.at[t],
                    out_hbm.at[pl.ds(base + (j0 + t) * CHUNK, CHUNK)])
            return carry

        lax.fori_loop(0, n_ch // kk, body, 0)

    return k(table, idx2d)


def _sc_scatter_add(rows, idx, n_out):
    n_rows, d = rows.shape
    per_w = n_rows // NW
    n_ch = per_w // CHUNK
    zeros = jnp.zeros((n_out, d), jnp.float32)
    mesh = plsc.VectorSubcoreMesh(core_axis_name="c", subcore_axis_name="s")

    @functools.partial(
        pl.kernel,
        out_type=jax.ShapeDtypeStruct((NC, n_out, d), jnp.float32),
        mesh=mesh,
        scratch_types=[
            pltpu.VMEM((CHUNK,), jnp.int32),
            pltpu.VMEM((CHUNK, d), jnp.float32),
            pltpu.VMEM_SHARED((n_out, d), jnp.float32),
        ],
    )
    def k(rows_hbm, idx_hbm, zeros_hbm, out_hbm, idx_v, rows_v, accum):
        cid = lax.axis_index("c")
        sid = lax.axis_index("s")
        wid = sid * NC + cid

        @pl.when(sid == 0)
        def _():
            pltpu.sync_copy(zeros_hbm, accum)

        plsc.subcore_barrier()

        def body(j, carry):
            off = wid * per_w + j * CHUNK
            pltpu.sync_copy(idx_hbm.at[pl.ds(off, CHUNK)], idx_v)
            pltpu.sync_copy(rows_hbm.at[pl.ds(off, CHUNK)], rows_v)
            pltpu.sync_copy(rows_v, accum.at[idx_v], add=True)
            return carry

        lax.fori_loop(0, n_ch, body, 0)
        plsc.subcore_barrier()

        @pl.when(sid == 0)
        def _():
            pltpu.sync_copy(accum, out_hbm.at[cid])

    return k(rows, idx, zeros)


_BLK = 256


def _emb_kernel(pgs_ref, pgd_ref, out_ref):
    step = np.float32(RCUT / (NBASIS + 1))
    vals = (lax.broadcasted_iota(jnp.int32, (1, NBASIS), 1)
            .astype(jnp.float32) + 1.0) * step
    vec = pgd_ref[:, :3] - pgs_ref[:, :3]
    d2 = jnp.sum(vec * vec, axis=1, keepdims=True)
    dist = jnp.sqrt(d2 + 1e-12)
    diff = (dist - vals) * (1.0 / step)

    def sus(t):
        safe = jnp.where(t > 0.0, t, 1.0)
        return jnp.where(t > 0.0, jnp.exp(-1.0 / safe), 0.0)

    c = np.float32(1.14136 * np.exp(2.0))
    out_ref[...] = c * sus(diff + 1.0) * sus(1.0 - diff)


def _tc_emb(pgs, pgd):
    e = pgs.shape[0]
    grid = e // _BLK
    return pl.pallas_call(
        _emb_kernel,
        grid=(grid,),
        in_specs=[
            pl.BlockSpec((_BLK, 128), lambda i: (i, 0)),
            pl.BlockSpec((_BLK, 128), lambda i: (i, 0)),
        ],
        out_specs=pl.BlockSpec((_BLK, NBASIS), lambda i: (i, 0)),
        out_shape=jax.ShapeDtypeStruct((e, NBASIS), jnp.float32),
    )(pgs, pgd)


def _layer_kernel(nin, nout, pad_to, packed, emb_ref, g_ref,
                  w1_ref, w2_ref, out_ref):
    emb = emb_ref[...]
    h = jnp.dot(emb, w1_ref[...], preferred_element_type=jnp.float32)
    h = jax.nn.silu(h) * np.float32(SILU_C)
    tpw = jnp.dot(h, w2_ref[...], preferred_element_type=jnp.float32)
    g = g_ref[...]
    if packed:
        xe = g[:, :nin] + g[:, 64:64 + nin]
        xe = jax.nn.silu(xe) * np.float32(SILU_C)
    else:
        xe = g[:, :nin]
    acc = jnp.zeros((emb.shape[0], 128), jnp.float32)
    for i in range(nin):
        acc = acc + xe[:, i:i + 1] * tpw[:, i * 128:(i + 1) * 128]
    out_ref[...] = acc


def _tc_layer(emb, g, w1s, w2s, nin, nout, blk, packed):
    e = emb.shape[0]
    grid = e // blk
    din = g.shape[1]
    body = functools.partial(_layer_kernel, nin, nout, 128, packed)
    in_specs = [
        pl.BlockSpec((blk, NBASIS), lambda i: (i, 0)),
        pl.BlockSpec((blk, din), lambda i: (i, 0)),
        pl.BlockSpec(w1s.shape, lambda i: (0, 0)),
        pl.BlockSpec(w2s.shape, lambda i: (0, 0)),
    ]
    return pl.pallas_call(
        body,
        grid=(grid,),
        in_specs=in_specs,
        out_specs=pl.BlockSpec((blk, 128), lambda i: (i, 0)),
        out_shape=jax.ShapeDtypeStruct((e, 128), jnp.float32),
    )(emb, g, w1s, w2s)


def _pack_kernel(p_ref, out_ref):
    out_ref[...] = jnp.concatenate(
        [p_ref[0, :, :64], p_ref[1, :, :64]], axis=1)


def _tc_pack(part):
    n = part.shape[1]
    return pl.pallas_call(
        _pack_kernel,
        out_shape=jax.ShapeDtypeStruct((n, 128), jnp.float32),
    )(part)


def _combine_kernel(q_ref, out_ref):
    out_ref[...] = (q_ref[0] + q_ref[1])[:, :64]


def _tc_combine(q):
    return pl.pallas_call(
        _combine_kernel,
        out_shape=jax.ShapeDtypeStruct((q.shape[1], 64), jnp.float32),
    )(q)


def kernel(x, pos, edge_index, batch, W1_0, W2_0, W1_1, W2_1, W1_2, W2_2,
           W1_3, W2_3):
    f32 = jnp.float32
    x = x.astype(f32)
    src = edge_index[0].astype(jnp.int32)
    dst = edge_index[1].astype(jnp.int32)

    ws = []
    for (w1, w2), (nin, nout) in zip(
            [(W1_0, W2_0), (W1_1, W2_1), (W1_2, W2_2), (W1_3, W2_3)],
            LAYER_DIMS):
        w1s = (w1 / np.sqrt(NBASIS)).astype(f32)
        w2s = (w2 / (np.sqrt(NHID) * np.sqrt(nin) * np.sqrt(NUM_NEIGH))
               ).astype(f32)
        w2p = jnp.pad(w2s.reshape(NHID, nin, nout),
                      ((0, 0), (0, 0), (0, 128 - nout)))
        ws.append((w1s, w2p.reshape(NHID, nin * 128)))

    pos_pad = jnp.pad(pos.astype(f32), ((0, 0), (0, 125)))

    pgs = _sc_gather(pos_pad, src)
    pgd = _sc_gather(pos_pad, dst)
    emb = _tc_emb(pgs, pgd)

    gx = _sc_gather(x, dst)
    ef = _tc_layer(emb, gx, ws[0][0], ws[0][1], 128, 40, 128, False)
    part = _sc_scatter_add(ef, src, N_NODES)

    for l in (1, 2, 3):
        nin, nout = LAYER_DIMS[l]
        packed = _tc_pack(part)
        g = _sc_gather(packed, dst)
        ef = _tc_layer(emb, g, ws[l][0], ws[l][1], nin, nout, 256, True)
        part = _sc_scatter_add(ef, src, N_NODES)

    rows = part.reshape(2 * N_NODES, 128)
    pad_rows = 2 * N_NODES % (NW * CHUNK)
    pad_rows = NW * CHUNK - pad_rows if pad_rows else 0
    rows = jnp.pad(rows, ((0, pad_rows), (0, 0)))
    bidx = jnp.concatenate([batch.astype(jnp.int32)] * 2 +
                           [jnp.zeros((pad_rows,), jnp.int32)])
    q = _sc_scatter_add(rows, bidx, NGRAPH)
    return _tc_combine(q)


# speedup vs baseline: 2.5602x; 1.0660x over previous
import functools

import jax
import jax.numpy as jnp
import numpy as np
from jax import lax
from jax.experimental import pallas as pl
from jax.experimental.pallas import tpu as pltpu
from jax.experimental.pallas import tpu_sc as plsc

N_NODES = 10000
N_EDGES = 320000
NBASIS = 20
NHID = 100
RCUT = 5.0
NUM_NEIGH = 32.0
NGRAPH = 256
LAYER_DIMS = [(128, 40), (40, 40), (40, 40), (40, 64)]

NC = 2
NS = 16
NW = NC * NS
CHUNK = 80


def _normalize2mom_silu():
    z = np.linspace(-12.0, 12.0, 240001)
    pdf = np.exp(-0.5 * z * z) / np.sqrt(2.0 * np.pi)
    s = z / (1.0 + np.exp(-z))
    m2 = np.trapz(s * s * pdf, z)
    return float(m2 ** -0.5)


SILU_C = _normalize2mom_silu()


def _pipe_k(n_ch):
    for cand in (5, 4, 2):
        if n_ch % cand == 0:
            return cand
    return 1


def _sc_gather(table, idx):
    n_rows, d = table.shape
    n_idx = idx.shape[0]
    per_w = n_idx // NW
    n_ch = per_w // CHUNK
    kk = _pipe_k(n_ch)
    idx2d = idx.reshape(NW, n_ch, CHUNK)
    mesh = plsc.VectorSubcoreMesh(core_axis_name="c", subcore_axis_name="s")

    @functools.partial(
        pl.kernel,
        out_type=jax.ShapeDtypeStruct((n_idx, d), jnp.float32),
        mesh=mesh,
        scratch_types=[
            pltpu.VMEM((n_ch, CHUNK), jnp.int32),
            pltpu.VMEM((kk, CHUNK, d), jnp.float32),
            pltpu.SemaphoreType.DMA,
        ],
    )
    def k(table_hbm, idx_hbm, out_hbm, idx_v, rows_v, sem):
        wid = lax.axis_index("s") * NC + lax.axis_index("c")
        base = wid * per_w
        pltpu.sync_copy(idx_hbm.at[wid], idx_v)

        def body(b, carry):
            j0 = b * kk
            hs = []
            for t in range(kk):
                hs.append(pltpu.async_copy(
                    table_hbm.at[idx_v.at[j0 + t]], rows_v.at[t], sem))
            for t in range(kk):
                hs[t].wait()
                pltpu.sync_copy(
                    rows_v.at[t],
                    out_hbm.at[pl.ds(base + (j0 + t) * CHUNK, CHUNK)])
            return carry

        lax.fori_loop(0, n_ch // kk, body, 0)

    return k(table, idx2d)


def _sc_scatter_add(rows, idx, n_out):
    n_rows, d = rows.shape
    per_w = n_rows // NW
    n_ch = per_w // CHUNK
    zeros = jnp.zeros((n_out, d), jnp.float32)
    mesh = plsc.VectorSubcoreMesh(core_axis_name="c", subcore_axis_name="s")

    @functools.partial(
        pl.kernel,
        out_type=jax.ShapeDtypeStruct((NC, n_out, d), jnp.float32),
        mesh=mesh,
        scratch_types=[
            pltpu.VMEM((CHUNK,), jnp.int32),
            pltpu.VMEM((CHUNK, d), jnp.float32),
            pltpu.VMEM_SHARED((n_out, d), jnp.float32),
        ],
    )
    def k(rows_hbm, idx_hbm, zeros_hbm, out_hbm, idx_v, rows_v, accum):
        cid = lax.axis_index("c")
        sid = lax.axis_index("s")
        wid = sid * NC + cid

        @pl.when(sid == 0)
        def _():
            pltpu.sync_copy(zeros_hbm, accum)

        plsc.subcore_barrier()

        def body(j, carry):
            off = wid * per_w + j * CHUNK
            pltpu.sync_copy(idx_hbm.at[pl.ds(off, CHUNK)], idx_v)
            pltpu.sync_copy(rows_hbm.at[pl.ds(off, CHUNK)], rows_v)
            pltpu.sync_copy(rows_v, accum.at[idx_v], add=True)
            return carry

        lax.fori_loop(0, n_ch, body, 0)
        plsc.subcore_barrier()

        @pl.when(sid == 0)
        def _():
            pltpu.sync_copy(accum, out_hbm.at[cid])

    return k(rows, idx, zeros)


_BLK = 256


def _emb_kernel(pgs_ref, pgd_ref, out_ref):
    step = np.float32(RCUT / (NBASIS + 1))
    vals = (lax.broadcasted_iota(jnp.int32, (1, NBASIS), 1)
            .astype(jnp.float32) + 1.0) * step
    vec = pgd_ref[:, :3] - pgs_ref[:, :3]
    d2 = jnp.sum(vec * vec, axis=1, keepdims=True)
    dist = jnp.sqrt(d2 + 1e-12)
    diff = (dist - vals) * (1.0 / step)

    def sus(t):
        safe = jnp.where(t > 0.0, t, 1.0)
        return jnp.where(t > 0.0, jnp.exp(-1.0 / safe), 0.0)

    c = np.float32(1.14136 * np.exp(2.0))
    out_ref[...] = c * sus(diff + 1.0) * sus(1.0 - diff)


def _tc_emb(pgs, pgd):
    e = pgs.shape[0]
    grid = e // _BLK
    return pl.pallas_call(
        _emb_kernel,
        grid=(grid,),
        in_specs=[
            pl.BlockSpec((_BLK, 128), lambda i: (i, 0)),
            pl.BlockSpec((_BLK, 128), lambda i: (i, 0)),
        ],
        out_specs=pl.BlockSpec((_BLK, NBASIS), lambda i: (i, 0)),
        out_shape=jax.ShapeDtypeStruct((e, NBASIS), jnp.float32),
    )(pgs, pgd)


def _layer_kernel(nin, nout, pad_to, packed, emb_ref, g_ref,
                  w1_ref, w2_ref, out_ref):
    emb = emb_ref[...]
    h = jnp.dot(emb, w1_ref[...], preferred_element_type=jnp.float32)
    h = jax.nn.silu(h) * np.float32(SILU_C)
    tpw = jnp.dot(h, w2_ref[...], preferred_element_type=jnp.float32)
    g = g_ref[...]
    if packed:
        xe = g[:, :nin] + g[:, 64:64 + nin]
        xe = jax.nn.silu(xe) * np.float32(SILU_C)
    else:
        xe = g[:, :nin]
    acc = jnp.zeros((emb.shape[0], 128), jnp.float32)
    for i in range(nin):
        acc = acc + xe[:, i:i + 1] * tpw[:, i * 128:(i + 1) * 128]
    out_ref[...] = acc


def _tc_layer(emb, g, w1s, w2s, nin, nout, blk, packed):
    e = emb.shape[0]
    grid = e // blk
    din = g.shape[1]
    body = functools.partial(_layer_kernel, nin, nout, 128, packed)
    in_specs = [
        pl.BlockSpec((blk, NBASIS), lambda i: (i, 0)),
        pl.BlockSpec((blk, din), lambda i: (i, 0)),
        pl.BlockSpec(w1s.shape, lambda i: (0, 0)),
        pl.BlockSpec(w2s.shape, lambda i: (0, 0)),
    ]
    return pl.pallas_call(
        body,
        grid=(grid,),
        in_specs=in_specs,
        out_specs=pl.BlockSpec((blk, 128), lambda i: (i, 0)),
        out_shape=jax.ShapeDtypeStruct((e, 128), jnp.float32),
    )(emb, g, w1s, w2s)


def _pack_kernel(p_ref, out_ref):
    out_ref[...] = jnp.concatenate(
        [p_ref[0, :, :64], p_ref[1, :, :64]], axis=1)


def _tc_pack(part):
    n = part.shape[1]
    return pl.pallas_call(
        _pack_kernel,
        out_shape=jax.ShapeDtypeStruct((n, 128), jnp.float32),
    )(part)


def _combine_kernel(q_ref, out_ref):
    out_ref[...] = (q_ref[0] + q_ref[1])[:, :64]


def _tc_combine(q):
    return pl.pallas_call(
        _combine_kernel,
        out_shape=jax.ShapeDtypeStruct((q.shape[1], 64), jnp.float32),
    )(q)


def kernel(x, pos, edge_index, batch, W1_0, W2_0, W1_1, W2_1, W1_2, W2_2,
           W1_3, W2_3):
    f32 = jnp.float32
    x = x.astype(f32)
    src = edge_index[0].astype(jnp.int32)
    dst = edge_index[1].astype(jnp.int32)

    ws = []
    for (w1, w2), (nin, nout) in zip(
            [(W1_0, W2_0), (W1_1, W2_1), (W1_2, W2_2), (W1_3, W2_3)],
            LAYER_DIMS):
        w1s = (w1 / np.sqrt(NBASIS)).astype(f32)
        w2s = (w2 / (np.sqrt(NHID) * np.sqrt(nin) * np.sqrt(NUM_NEIGH))
               ).astype(f32)
        w2p = jnp.pad(w2s.reshape(NHID, nin, nout),
                      ((0, 0), (0, 0), (0, 128 - nout)))
        ws.append((w1s, w2p.reshape(NHID, nin * 128)))

    pos_pad = jnp.pad(pos.astype(f32), ((0, 0), (0, 125)))

    pgs = _sc_gather(pos_pad, src)
    pgd = _sc_gather(pos_pad, dst)
    emb = _tc_emb(pgs, pgd)

    gx = _sc_gather(x, dst)
    ef = _tc_layer(emb, gx, ws[0][0], ws[0][1], 128, 40, 256, False)
    part = _sc_scatter_add(ef, src, N_NODES)

    for l in (1, 2, 3):
        nin, nout = LAYER_DIMS[l]
        packed = _tc_pack(part)
        g = _sc_gather(packed, dst)
        ef = _tc_layer(emb, g, ws[l][0], ws[l][1], nin, nout, 512, True)
        part = _sc_scatter_add(ef, src, N_NODES)

    rows = part.reshape(2 * N_NODES, 128)
    pad_rows = 2 * N_NODES % (NW * CHUNK)
    pad_rows = NW * CHUNK - pad_rows if pad_rows else 0
    rows = jnp.pad(rows, ((0, pad_rows), (0, 0)))
    bidx = jnp.concatenate([batch.astype(jnp.int32)] * 2 +
                           [jnp.zeros((pad_rows,), jnp.int32)])
    q = _sc_scatter_add(rows, bidx, NGRAPH)
    return _tc_combine(q)
